# Initial kernel scaffold; baseline (speedup 1.0000x reference)
import jax
import jax.numpy as jnp
from jax.experimental import pallas as pl


def kernel(x, edge_index, batch, W1, b1, g1, be1, W2, b2, g2, be2, Wg1, bg1, Wg2, bg2, Wc1, bc1, Wc2, bc2):
    raise NotImplementedError("write your pallas kernel here")


# speedup vs baseline: 10.8870x; 10.8870x over previous
import functools

import jax
import jax.numpy as jnp
from jax import lax
from jax.experimental import pallas as pl
from jax.experimental.pallas import tpu as pltpu
import jax.experimental.pallas.tpu_sc as plsc

_N = 10000
_E = 320000
_D = 128
_H = 256
_PH = 128
_G = 64

_NB = 512
_NPAD = 10240
_NBLK = _NPAD // _NB
_CHUNK = 128
_TILES = 16
_CH = 157
_EPAD = _TILES * _CHUNK * _CH
_CHD = 79
_EPADD = 32 * _CHUNK * _CHD
_RPT = _NPAD // _TILES
_HC = _H // 2


def _deg_body(dst_hbm, out_hbm, deg_sh, ones_v, zeros_v, idx_v):
    cid = lax.axis_index("c")
    sid = lax.axis_index("s")
    wid = cid * _TILES + sid

    ones16 = jnp.ones((16,), jnp.float32)
    zero16 = jnp.zeros((16,), jnp.float32)

    def fill(i, _):
        for j in range(_HC // 16):
            ones_v[i, pl.ds(j * 16, 16)] = ones16
            zeros_v[i, pl.ds(j * 16, 16)] = zero16
        return 0

    lax.fori_loop(0, _CHUNK, fill, 0)

    def zero(i, _):
        pltpu.sync_copy(zeros_v, deg_sh.at[pl.ds(sid * _RPT + i * _CHUNK, _CHUNK)])
        return 0

    lax.fori_loop(0, _RPT // _CHUNK, zero, 0)
    plsc.subcore_barrier()

    def chunk(ch, _):
        base = wid * (_CHD * _CHUNK) + ch * _CHUNK
        pltpu.sync_copy(dst_hbm.at[pl.ds(base, _CHUNK)], idx_v)
        pltpu.sync_copy(ones_v, deg_sh.at[idx_v], add=True)
        return 0

    lax.fori_loop(0, _CHD, chunk, 0)
    plsc.subcore_barrier()

    def wout(i, _):
        r0 = sid * _RPT + i * _CHUNK
        pltpu.sync_copy(deg_sh.at[pl.ds(r0, _CHUNK)],
                        out_hbm.at[cid, pl.ds(r0, _CHUNK)])
        return 0

    lax.fori_loop(0, _RPT // _CHUNK, wout, 0)


@functools.lru_cache(maxsize=1)
def _sc_kernels():
    mesh = plsc.VectorSubcoreMesh(
        core_axis_name="c", subcore_axis_name="s",
        num_cores=2, num_subcores=16,
    )
    deg_kernel = pl.kernel(
        _deg_body,
        out_type=jax.ShapeDtypeStruct((2, _NPAD, _HC), jnp.float32),
        mesh=mesh,
        scratch_types=[
            pltpu.VMEM_SHARED((_NPAD, _HC), jnp.float32),
            pltpu.VMEM((_CHUNK, _HC), jnp.float32),
            pltpu.VMEM((_CHUNK, _HC), jnp.float32),
            pltpu.VMEM((_CHUNK,), jnp.int32),
        ],
    )
    agg_kernel = pl.kernel(
        _agg_body,
        out_type=jax.ShapeDtypeStruct((2, _NPAD, _HC), jnp.float32),
        mesh=mesh,
        scratch_types=[
            pltpu.VMEM_SHARED((_NPAD, _HC), jnp.float32),
            pltpu.VMEM((_CHUNK, _HC), jnp.float32),
            pltpu.VMEM((_CHUNK,), jnp.int32),
            pltpu.VMEM((_CHUNK,), jnp.int32),
            pltpu.VMEM((_CHUNK, _HC), jnp.float32),
            pltpu.SemaphoreType.DMA,
        ],
    )
    return deg_kernel, agg_kernel


def _agg_body(u0_hbm, u1_hbm, src_hbm, dst_hbm, out_hbm,
              acc_sh, zeros_v, sidx_v, didx_v, rows_v, sem):
    cid = lax.axis_index("c")
    sid = lax.axis_index("s")

    zero16 = jnp.zeros((16,), jnp.float32)

    def fill(i, _):
        for j in range(_HC // 16):
            zeros_v[i, pl.ds(j * 16, 16)] = zero16
        return 0

    lax.fori_loop(0, _CHUNK, fill, 0)

    def zero(i, _):
        pltpu.sync_copy(zeros_v, acc_sh.at[pl.ds(sid * _RPT + i * _CHUNK, _CHUNK)])
        return 0

    lax.fori_loop(0, _RPT // _CHUNK, zero, 0)
    plsc.subcore_barrier()

    def chunk(ch, _):
        base = sid * (_CH * _CHUNK) + ch * _CHUNK
        pltpu.sync_copy(src_hbm.at[pl.ds(base, _CHUNK)], sidx_v)
        pltpu.sync_copy(dst_hbm.at[pl.ds(base, _CHUNK)], didx_v)

        @pl.when(cid == 0)
        def _():
            pltpu.async_copy(u0_hbm.at[sidx_v], rows_v, sem).wait()

        @pl.when(cid == 1)
        def _():
            pltpu.async_copy(u1_hbm.at[sidx_v], rows_v, sem).wait()

        pltpu.sync_copy(rows_v, acc_sh.at[didx_v], add=True)
        return 0

    lax.fori_loop(0, _CH, chunk, 0)
    plsc.subcore_barrier()

    def wout(i, _):
        r0 = sid * _RPT + i * _CHUNK
        pltpu.sync_copy(acc_sh.at[pl.ds(r0, _CHUNK)],
                        out_hbm.at[cid, pl.ds(r0, _CHUNK)])
        return 0

    lax.fori_loop(0, _RPT // _CHUNK, wout, 0)


def _t1_body(x_ref, deg_ref, w1_ref, u0_ref, u1_ref):
    deg = deg_ref[0, :, 0:1] + deg_ref[1, :, 0:1]
    dis = lax.rsqrt(deg + 1.0)
    h = jnp.dot(x_ref[...], w1_ref[...], preferred_element_type=jnp.float32)
    u = h * dis
    u0_ref[...] = u[:, :_HC]
    u1_ref[...] = u[:, _HC:]


def _t2_body(y_ref, u0_ref, u1_ref, deg_ref, b1_ref, g1_ref, be1_ref, w2_ref,
             o0_ref, o1_ref):
    deg = deg_ref[0, :, 0:1] + deg_ref[1, :, 0:1]
    dis = lax.rsqrt(deg + 1.0)
    y = jnp.concatenate(
        [y_ref[0] + u0_ref[...], y_ref[1] + u1_ref[...]], axis=-1)
    t = y * dis + b1_ref[...]
    mu = jnp.mean(t, axis=-1, keepdims=True)
    var = jnp.mean((t - mu) ** 2, axis=-1, keepdims=True)
    t = (t - mu) * lax.rsqrt(var + 1e-5) * g1_ref[...] + be1_ref[...]
    t = jnp.maximum(t, 0.0)
    h2 = jnp.dot(t, w2_ref[...], preferred_element_type=jnp.float32)
    u = h2 * dis
    o0_ref[...] = u[:, :_HC]
    o1_ref[...] = u[:, _HC:]


def _t3_body(y_ref, u0_ref, u1_ref, deg_ref, batch_ref, b2_ref, g2_ref,
             be2_ref, wg1_ref, bg1_ref, wg2_ref, bg2_ref, wc1_ref, bc1_ref,
             wc2_ref, bc2_ref, out_ref, m_scr, den_scr, num_scr):
    i = pl.program_id(0)

    @pl.when(i == 0)
    def _():
        m_scr[...] = jnp.full_like(m_scr[...], -jnp.inf)
        den_scr[...] = jnp.zeros_like(den_scr[...])
        num_scr[...] = jnp.zeros_like(num_scr[...])

    deg = deg_ref[0, :, 0:1] + deg_ref[1, :, 0:1]
    dis = lax.rsqrt(deg + 1.0)
    y = jnp.concatenate(
        [y_ref[0] + u0_ref[...], y_ref[1] + u1_ref[...]], axis=-1)
    t = y * dis + b2_ref[...]
    mu = jnp.mean(t, axis=-1, keepdims=True)
    var = jnp.mean((t - mu) ** 2, axis=-1, keepdims=True)
    h = (t - mu) * lax.rsqrt(var + 1e-5) * g2_ref[...] + be2_ref[...]
    h = jnp.maximum(h, 0.0)

    z = jnp.maximum(
        jnp.dot(h, wg1_ref[...], preferred_element_type=jnp.float32)
        + bg1_ref[...], 0.0)
    gate = (jnp.dot(z, wg2_ref[...], preferred_element_type=jnp.float32)
            + bg2_ref[0, 0])

    b = batch_ref[0, 0, :]
    seg = lax.broadcasted_iota(jnp.int32, (1, 128), 1)
    onehot = b[:, None] == seg
    gmat = jnp.where(onehot, gate, -jnp.inf)
    bmax = jnp.max(gmat, axis=0, keepdims=True)

    m_old = m_scr[0:1, :]
    m_new = jnp.maximum(m_old, bmax)
    scale = jnp.where(m_old == -jnp.inf, 0.0, jnp.exp(m_old - m_new))
    e = jnp.where(onehot, jnp.exp(gmat - m_new), 0.0)

    den_scr[0:1, :] = den_scr[0:1, :] * scale + jnp.sum(e, axis=0, keepdims=True)
    num_scr[...] = num_scr[...] * scale + lax.dot_general(
        h, e, (((0,), (0,)), ((), ())),
        preferred_element_type=jnp.float32)
    m_scr[0:1, :] = m_new

    @pl.when(i == _NBLK - 1)
    def _():
        den = den_scr[0:1, :]
        pooled = jnp.where(den > 0.0, num_scr[...] / den, 0.0)
        zz = jnp.maximum(
            lax.dot_general(pooled, wc1_ref[...], (((0,), (0,)), ((), ())),
                            preferred_element_type=jnp.float32)
            + bc1_ref[...], 0.0)
        res = (jnp.dot(zz, wc2_ref[...], preferred_element_type=jnp.float32)
               + bc2_ref[...])
        out_ref[...] = res[:_G, :]


def _full(deg_spec, row_spec):
    return deg_spec, row_spec


def kernel(x, edge_index, batch, W1, b1, g1, be1, W2, b2, g2, be2,
           Wg1, bg1, Wg2, bg2, Wc1, bc1, Wc2, bc2):
    f32 = jnp.float32
    x_pad = jnp.pad(x, ((0, _NPAD - _N), (0, 0)))
    batch_pad = jnp.pad(batch.astype(jnp.int32), (0, _NPAD - _N),
                        constant_values=_G)
    batch_r = batch_pad.reshape(_NBLK, 1, _NB)

    src = edge_index[0].astype(jnp.int32)
    dst = edge_index[1].astype(jnp.int32)
    npad_e = _EPAD - _E
    pad_idx = _N + (jnp.arange(npad_e, dtype=jnp.int32) % (_NPAD - _N))
    src_p = jnp.concatenate([src, pad_idx])
    dst_p = jnp.concatenate([dst, pad_idx])
    npad_d = _EPADD - _E
    pad_idx_d = _N + (jnp.arange(npad_d, dtype=jnp.int32) % (_NPAD - _N))
    dst_d = jnp.concatenate([dst, pad_idx_d])

    deg_kernel, agg_kernel = _sc_kernels()
    deg = deg_kernel(dst_d)

    row = lambda i: (i, 0)
    c3 = lambda i: (0, i, 0)
    const2 = lambda i: (0, 0)

    u1a, u1b = pl.pallas_call(
        _t1_body,
        grid=(_NBLK,),
        in_specs=[
            pl.BlockSpec((_NB, _D), row),
            pl.BlockSpec((2, _NB, _HC), c3),
            pl.BlockSpec((_D, _H), const2),
        ],
        out_specs=[
            pl.BlockSpec((_NB, _HC), row),
            pl.BlockSpec((_NB, _HC), row),
        ],
        out_shape=[
            jax.ShapeDtypeStruct((_NPAD, _HC), f32),
            jax.ShapeDtypeStruct((_NPAD, _HC), f32),
        ],
    )(x_pad, deg, W1)

    y1 = agg_kernel(u1a, u1b, src_p, dst_p)

    u2a, u2b = pl.pallas_call(
        _t2_body,
        grid=(_NBLK,),
        in_specs=[
            pl.BlockSpec((2, _NB, _HC), c3),
            pl.BlockSpec((_NB, _HC), row),
            pl.BlockSpec((_NB, _HC), row),
            pl.BlockSpec((2, _NB, _HC), c3),
            pl.BlockSpec((1, _H), const2),
            pl.BlockSpec((1, _H), const2),
            pl.BlockSpec((1, _H), const2),
            pl.BlockSpec((_H, _H), const2),
        ],
        out_specs=[
            pl.BlockSpec((_NB, _HC), row),
            pl.BlockSpec((_NB, _HC), row),
        ],
        out_shape=[
            jax.ShapeDtypeStruct((_NPAD, _HC), f32),
            jax.ShapeDtypeStruct((_NPAD, _HC), f32),
        ],
    )(y1, u1a, u1b, deg, b1.reshape(1, _H), g1.reshape(1, _H),
      be1.reshape(1, _H), W2)

    y2 = agg_kernel(u2a, u2b, src_p, dst_p)

    out = pl.pallas_call(
        _t3_body,
        grid=(_NBLK,),
        in_specs=[
            pl.BlockSpec((2, _NB, _HC), c3),
            pl.BlockSpec((_NB, _HC), row),
            pl.BlockSpec((_NB, _HC), row),
            pl.BlockSpec((2, _NB, _HC), c3),
            pl.BlockSpec((1, 1, _NB), lambda i: (i, 0, 0)),
            pl.BlockSpec((1, _H), const2),
            pl.BlockSpec((1, _H), const2),
            pl.BlockSpec((1, _H), const2),
            pl.BlockSpec((_H, _PH), const2),
            pl.BlockSpec((1, _PH), const2),
            pl.BlockSpec((_PH, 1), const2),
            pl.BlockSpec((1, 1), const2),
            pl.BlockSpec((_H, _PH), const2),
            pl.BlockSpec((1, _PH), const2),
            pl.BlockSpec((_PH, 2), const2),
            pl.BlockSpec((1, 2), const2),
        ],
        out_specs=pl.BlockSpec((_G, 2), const2),
        out_shape=jax.ShapeDtypeStruct((_G, 2), f32),
        scratch_shapes=[
            pltpu.VMEM((8, 128), f32),
            pltpu.VMEM((8, 128), f32),
            pltpu.VMEM((_H, 128), f32),
        ],
    )(y2, u2a, u2b, deg, batch_r, b2.reshape(1, _H), g2.reshape(1, _H),
      be2.reshape(1, _H), Wg1, bg1.reshape(1, _PH), Wg2, bg2.reshape(1, 1),
      Wc1, bc1.reshape(1, _PH), Wc2, bc2.reshape(1, 2))

    return out
